# initial kernel scaffold (unmeasured)
import jax
import jax.numpy as jnp
from jax import lax
from jax.experimental import pallas as pl
from jax.experimental.pallas import tpu as pltpu

N_DEV = 4


def kernel(x, w_mat):
    m_per, k = x.shape
    _, n_per = w_mat.shape
    half = m_per // 2

    def body(x_hbm, w_ref, out_ref, xg_hbm, stage_ref, amax_ref,
             ssems, rsems, asend, arecv, lsems):
        my = lax.axis_index("i")
        left = lax.rem(my + N_DEV - 1, N_DEV)
        right = lax.rem(my + 1, N_DEV)
        diag = lax.rem(my + 2, N_DEV)

        bsem = pltpu.get_barrier_semaphore()
        for nbr in (left, right):
            pl.semaphore_signal(bsem, inc=1, device_id=(nbr,),
                                device_id_type=pl.DeviceIdType.MESH)
        pl.semaphore_wait(bsem, 2)

        h1r = pltpu.make_async_remote_copy(
            src_ref=x_hbm, dst_ref=xg_hbm.at[0],
            send_sem=ssems.at[0], recv_sem=rsems.at[0],
            device_id=(right,), device_id_type=pl.DeviceIdType.MESH)
        h1l = pltpu.make_async_remote_copy(
            src_ref=x_hbm, dst_ref=xg_hbm.at[1],
            send_sem=ssems.at[1], recv_sem=rsems.at[1],
            device_id=(left,), device_id_type=pl.DeviceIdType.MESH)
        h1r.start()
        h1l.start()

        w_bf = w_ref[...].astype(jnp.bfloat16)

        def mm(a):
            return lax.dot_general(
                a.astype(jnp.bfloat16), w_bf,
                (((1,), (0,)), ((), ())),
                preferred_element_type=jnp.float32)

        def compute_chunk(src_hbm, block, amax_acc):
            c0 = pltpu.make_async_copy(
                src_hbm.at[pl.ds(0, half)], stage_ref.at[0], lsems.at[0])
            c1 = pltpu.make_async_copy(
                src_hbm.at[pl.ds(half, half)], stage_ref.at[1], lsems.at[1])
            c0.start()
            c1.start()
            c0.wait()
            y0 = mm(stage_ref[0])
            out_ref[pl.ds(block * m_per, half), :] = y0
            c1.wait()
            y1 = mm(stage_ref[1])
            out_ref[pl.ds(block * m_per + half, half), :] = y1
            m01 = jnp.maximum(jnp.max(jnp.abs(y0)), jnp.max(jnp.abs(y1)))
            return jnp.maximum(amax_acc, m01)

        amax_loc = compute_chunk(x_hbm, my, jnp.float32(0.0))

        h1r.wait()
        h1l.wait()

        h2r = pltpu.make_async_remote_copy(
            src_ref=xg_hbm.at[0, pl.ds(0, half)],
            dst_ref=xg_hbm.at[2, pl.ds(0, half)],
            send_sem=ssems.at[2], recv_sem=rsems.at[2],
            device_id=(right,), device_id_type=pl.DeviceIdType.MESH)
        h2l = pltpu.make_async_remote_copy(
            src_ref=xg_hbm.at[1, pl.ds(half, half)],
            dst_ref=xg_hbm.at[2, pl.ds(half, half)],
            send_sem=ssems.at[3], recv_sem=rsems.at[3],
            device_id=(left,), device_id_type=pl.DeviceIdType.MESH)
        h2r.start()
        h2l.start()

        amax_loc = compute_chunk(xg_hbm.at[0], left, amax_loc)
        amax_loc = compute_chunk(xg_hbm.at[1], right, amax_loc)

        h2r.wait()
        h2l.wait()

        amax_loc = compute_chunk(xg_hbm.at[2], diag, amax_loc)

        amax_ref[my] = jnp.full((8, 128), amax_loc, jnp.float32)
        sends = []
        for j, tgt in enumerate((left, right, diag)):
            s = pltpu.make_async_remote_copy(
                src_ref=amax_ref.at[my], dst_ref=amax_ref.at[my],
                send_sem=asend.at[j], recv_sem=arecv.at[my],
                device_id=(tgt,), device_id_type=pl.DeviceIdType.MESH)
            s.start()
            sends.append(s)
        for src_dev in (left, right, diag):
            r = pltpu.make_async_remote_copy(
                src_ref=amax_ref.at[src_dev], dst_ref=amax_ref.at[src_dev],
                send_sem=asend.at[3], recv_sem=arecv.at[src_dev],
                device_id=(src_dev,), device_id_type=pl.DeviceIdType.MESH)
            r.wait_recv()
        for s in sends:
            s.wait_send()

        g_amax = jnp.max(amax_ref[...])
        scale = g_amax / 127.0
        q = jnp.clip(jnp.round(out_ref[...] / scale), -127.0, 127.0)
        out_ref[...] = q * scale

    return pl.pallas_call(
        body,
        out_shape=jax.ShapeDtypeStruct((N_DEV * m_per, n_per), jnp.float32),
        in_specs=[
            pl.BlockSpec(memory_space=pl.ANY),
            pl.BlockSpec(memory_space=pltpu.VMEM),
        ],
        out_specs=pl.BlockSpec(memory_space=pltpu.VMEM),
        scratch_shapes=[
            pl.ANY((N_DEV - 1, m_per, k), jnp.float32),
            pltpu.VMEM((2, half, k), jnp.float32),
            pltpu.VMEM((N_DEV, 8, 128), jnp.float32),
            pltpu.SemaphoreType.DMA((4,)),
            pltpu.SemaphoreType.DMA((4,)),
            pltpu.SemaphoreType.DMA((4,)),
            pltpu.SemaphoreType.DMA((4,)),
            pltpu.SemaphoreType.DMA((2,)),
        ],
        compiler_params=pltpu.CompilerParams(collective_id=0),
    )(x, w_mat)


# baseline (device time: 298083 ns/iter reference)
import jax
import jax.numpy as jnp
from jax import lax
from jax.experimental import pallas as pl
from jax.experimental.pallas import tpu as pltpu

N_DEV = 4


def kernel(x, w_mat):
    m_per, k = x.shape
    _, n_per = w_mat.shape
    half = m_per // 2

    def body(x_hbm, w_ref, out_ref, xg_hbm, stage_ref, amax_ref,
             ssems, rsems, asend, arecv, lsems):
        my = lax.axis_index("i")
        left = lax.rem(my + N_DEV - 1, N_DEV)
        right = lax.rem(my + 1, N_DEV)
        diag = lax.rem(my + 2, N_DEV)

        bsem = pltpu.get_barrier_semaphore()
        for nbr in (left, right):
            pl.semaphore_signal(bsem, inc=1, device_id=(nbr,),
                                device_id_type=pl.DeviceIdType.MESH)
        pl.semaphore_wait(bsem, 2)

        h1r = pltpu.make_async_remote_copy(
            src_ref=x_hbm, dst_ref=xg_hbm.at[0],
            send_sem=ssems.at[0], recv_sem=rsems.at[0],
            device_id=(right,), device_id_type=pl.DeviceIdType.MESH)
        h1l = pltpu.make_async_remote_copy(
            src_ref=x_hbm, dst_ref=xg_hbm.at[1],
            send_sem=ssems.at[1], recv_sem=rsems.at[1],
            device_id=(left,), device_id_type=pl.DeviceIdType.MESH)
        h1r.start()
        h1l.start()

        w_bf = w_ref[...].astype(jnp.bfloat16)

        def mm(a):
            return lax.dot_general(
                a.astype(jnp.bfloat16), w_bf,
                (((1,), (0,)), ((), ())),
                preferred_element_type=jnp.float32)

        def compute_chunk(src_hbm, block, amax_acc):
            c0 = pltpu.make_async_copy(
                src_hbm.at[pl.ds(0, half)], stage_ref.at[0], lsems.at[0])
            c1 = pltpu.make_async_copy(
                src_hbm.at[pl.ds(half, half)], stage_ref.at[1], lsems.at[1])
            c0.start()
            c1.start()
            c0.wait()
            y0 = mm(stage_ref[0])
            out_ref[pl.ds(block * m_per, half), :] = y0
            c1.wait()
            y1 = mm(stage_ref[1])
            out_ref[pl.ds(block * m_per + half, half), :] = y1
            m01 = jnp.maximum(jnp.max(jnp.abs(y0)), jnp.max(jnp.abs(y1)))
            return jnp.maximum(amax_acc, m01)

        amax_loc = compute_chunk(x_hbm, my, jnp.float32(0.0))

        h1r.wait()
        h1l.wait()

        h2r = pltpu.make_async_remote_copy(
            src_ref=xg_hbm.at[0, pl.ds(0, half)],
            dst_ref=xg_hbm.at[2, pl.ds(0, half)],
            send_sem=ssems.at[2], recv_sem=rsems.at[2],
            device_id=(right,), device_id_type=pl.DeviceIdType.MESH)
        h2l = pltpu.make_async_remote_copy(
            src_ref=xg_hbm.at[1, pl.ds(half, half)],
            dst_ref=xg_hbm.at[2, pl.ds(half, half)],
            send_sem=ssems.at[3], recv_sem=rsems.at[3],
            device_id=(left,), device_id_type=pl.DeviceIdType.MESH)
        h2r.start()
        h2l.start()

        amax_loc = compute_chunk(xg_hbm.at[0], left, amax_loc)
        amax_loc = compute_chunk(xg_hbm.at[1], right, amax_loc)

        h2r.wait()
        h2l.wait()

        amax_loc = compute_chunk(xg_hbm.at[2], diag, amax_loc)

        amax_ref[my] = jnp.full((8, 128), amax_loc, jnp.float32)
        sends = []
        for j, tgt in enumerate((left, right, diag)):
            s = pltpu.make_async_remote_copy(
                src_ref=amax_ref.at[my], dst_ref=amax_ref.at[my],
                send_sem=asend.at[j], recv_sem=arecv.at[my],
                device_id=(tgt,), device_id_type=pl.DeviceIdType.MESH)
            s.start()
            sends.append(s)
        for src_dev in (left, right, diag):
            r = pltpu.make_async_remote_copy(
                src_ref=amax_ref.at[src_dev], dst_ref=amax_ref.at[src_dev],
                send_sem=asend.at[3], recv_sem=arecv.at[src_dev],
                device_id=(src_dev,), device_id_type=pl.DeviceIdType.MESH)
            r.wait_recv()
        for s in sends:
            s.wait_send()

        g_amax = jnp.max(amax_ref[...])
        scale = g_amax / 127.0
        q = jnp.clip(jnp.round(out_ref[...] / scale), -127.0, 127.0)
        out_ref[...] = q * scale

    out, _ = pl.pallas_call(
        body,
        out_shape=(
            jax.ShapeDtypeStruct((N_DEV * m_per, n_per), jnp.float32),
            jax.ShapeDtypeStruct((N_DEV - 1, m_per, k), jnp.float32),
        ),
        in_specs=[
            pl.BlockSpec(memory_space=pl.ANY),
            pl.BlockSpec(memory_space=pltpu.VMEM),
        ],
        out_specs=(
            pl.BlockSpec(memory_space=pltpu.VMEM),
            pl.BlockSpec(memory_space=pl.ANY),
        ),
        scratch_shapes=[
            pltpu.VMEM((2, half, k), jnp.float32),
            pltpu.VMEM((N_DEV, 8, 128), jnp.float32),
            pltpu.SemaphoreType.DMA((4,)),
            pltpu.SemaphoreType.DMA((4,)),
            pltpu.SemaphoreType.DMA((4,)),
            pltpu.SemaphoreType.DMA((4,)),
            pltpu.SemaphoreType.DMA((2,)),
        ],
        compiler_params=pltpu.CompilerParams(collective_id=0),
    )(x, w_mat)
    return out


# device time: 124698 ns/iter; 2.3904x vs baseline; 2.3904x over previous
import jax
import jax.numpy as jnp
from jax import lax
from jax.experimental import pallas as pl
from jax.experimental.pallas import tpu as pltpu

N_DEV = 4


def kernel(x, w_mat):
    m_per, k = x.shape
    _, n_per = w_mat.shape
    kh = k // 2
    mh = m_per // 2

    def body(x_hbm, w_ref, out_ref, wg, xbf, stage, yg, ysnd, amax_ref,
             wssem, wrsem, yssem, yrsem, asend, arecv, lsem):
        my = lax.axis_index("i")
        left = lax.rem(my + N_DEV - 1, N_DEV)
        right = lax.rem(my + 1, N_DEV)
        diag = lax.rem(my + 2, N_DEV)
        MESH = pl.DeviceIdType.MESH

        bsem = pltpu.get_barrier_semaphore()
        for nbr in (left, right, diag):
            pl.semaphore_signal(bsem, inc=1, device_id=(nbr,),
                                device_id_type=MESH)
        pl.semaphore_wait(bsem, 3)

        wg[my] = w_ref[...].astype(jnp.bfloat16)
        h1r = pltpu.make_async_remote_copy(
            src_ref=wg.at[my], dst_ref=wg.at[my],
            send_sem=wssem.at[0], recv_sem=wrsem.at[0],
            device_id=(right,), device_id_type=MESH)
        h1l = pltpu.make_async_remote_copy(
            src_ref=wg.at[my], dst_ref=wg.at[my],
            send_sem=wssem.at[1], recv_sem=wrsem.at[1],
            device_id=(left,), device_id_type=MESH)
        h1r.start()
        h1l.start()

        for t in range(2):
            cp = pltpu.make_async_copy(
                x_hbm.at[pl.ds(t * mh, mh)], stage, lsem)
            cp.start()
            cp.wait()
            xbf[pl.ds(t * mh, mh), :] = stage[...].astype(jnp.bfloat16)

        def mm_block(w_slot):
            wv = wg[w_slot]
            return [
                lax.dot_general(
                    xbf[pl.ds(t * mh, mh), :], wv,
                    (((1,), (0,)), ((), ())),
                    preferred_element_type=jnp.float32)
                for t in range(2)
            ]

        for t, yt in enumerate(mm_block(my)):
            yg[my, pl.ds(t * mh, mh), :] = yt.astype(jnp.bfloat16)

        h1r.wait()
        h1l.wait()

        h2r = pltpu.make_async_remote_copy(
            src_ref=wg.at[left, pl.ds(0, kh)],
            dst_ref=wg.at[left, pl.ds(0, kh)],
            send_sem=wssem.at[2], recv_sem=wrsem.at[2],
            device_id=(right,), device_id_type=MESH)
        h2l = pltpu.make_async_remote_copy(
            src_ref=wg.at[right, pl.ds(kh, kh)],
            dst_ref=wg.at[right, pl.ds(kh, kh)],
            send_sem=wssem.at[3], recv_sem=wrsem.at[3],
            device_id=(left,), device_id_type=MESH)
        h2r.start()
        h2l.start()

        for t, yt in enumerate(mm_block(left)):
            ysnd[0, pl.ds(t * mh, mh), :] = yt.astype(jnp.bfloat16)
        ysl = pltpu.make_async_remote_copy(
            src_ref=ysnd.at[0], dst_ref=yg.at[my],
            send_sem=yssem.at[0], recv_sem=yrsem.at[my],
            device_id=(left,), device_id_type=MESH)
        ysl.start()
        for t, yt in enumerate(mm_block(right)):
            ysnd[1, pl.ds(t * mh, mh), :] = yt.astype(jnp.bfloat16)
        ysr = pltpu.make_async_remote_copy(
            src_ref=ysnd.at[1], dst_ref=yg.at[my],
            send_sem=yssem.at[1], recv_sem=yrsem.at[my],
            device_id=(right,), device_id_type=MESH)
        ysr.start()

        h2r.wait()
        h2l.wait()

        for t, yt in enumerate(mm_block(diag)):
            ysnd[2, pl.ds(t * mh, mh), :] = yt.astype(jnp.bfloat16)
        ysd = pltpu.make_async_remote_copy(
            src_ref=ysnd.at[2], dst_ref=yg.at[my],
            send_sem=yssem.at[2], recv_sem=yrsem.at[my],
            device_id=(diag,), device_id_type=MESH)
        ysd.start()

        for s in (left, right, diag):
            rv = pltpu.make_async_remote_copy(
                src_ref=yg.at[s], dst_ref=yg.at[s],
                send_sem=yssem.at[3], recv_sem=yrsem.at[s],
                device_id=(s,), device_id_type=MESH)
            rv.wait_recv()

        am = jnp.float32(0.0)
        for j in range(N_DEV):
            am = jnp.maximum(am, jnp.max(jnp.abs(yg[j]).astype(jnp.float32)))

        amax_ref[my] = jnp.full((8, 128), am, jnp.float32)
        asends = []
        for j, tgt in enumerate((left, right, diag)):
            s = pltpu.make_async_remote_copy(
                src_ref=amax_ref.at[my], dst_ref=amax_ref.at[my],
                send_sem=asend.at[j], recv_sem=arecv.at[my],
                device_id=(tgt,), device_id_type=MESH)
            s.start()
            asends.append(s)
        for src_dev in (left, right, diag):
            rv = pltpu.make_async_remote_copy(
                src_ref=amax_ref.at[src_dev], dst_ref=amax_ref.at[src_dev],
                send_sem=asend.at[3], recv_sem=arecv.at[src_dev],
                device_id=(src_dev,), device_id_type=MESH)
            rv.wait_recv()

        g_amax = jnp.max(amax_ref[...])
        scale = g_amax / 127.0
        for j in range(N_DEV):
            blk = yg[j].astype(jnp.float32)
            q = jnp.clip(jnp.round(blk / scale), -127.0, 127.0)
            out_ref[pl.ds(j * m_per, m_per), :] = q * scale

        ysl.wait_send()
        ysr.wait_send()
        ysd.wait_send()
        for s in asends:
            s.wait_send()

    out, _, _ = pl.pallas_call(
        body,
        out_shape=(
            jax.ShapeDtypeStruct((N_DEV * m_per, n_per), jnp.float32),
            jax.ShapeDtypeStruct((N_DEV, k, n_per), jnp.bfloat16),
            jax.ShapeDtypeStruct((m_per, k), jnp.bfloat16),
        ),
        in_specs=[
            pl.BlockSpec(memory_space=pl.ANY),
            pl.BlockSpec(memory_space=pltpu.VMEM),
        ],
        out_specs=(
            pl.BlockSpec(memory_space=pltpu.VMEM),
            pl.BlockSpec(memory_space=pltpu.VMEM),
            pl.BlockSpec(memory_space=pltpu.VMEM),
        ),
        scratch_shapes=[
            pltpu.VMEM((mh, k), jnp.float32),
            pltpu.VMEM((N_DEV, m_per, n_per), jnp.bfloat16),
            pltpu.VMEM((3, m_per, n_per), jnp.bfloat16),
            pltpu.VMEM((N_DEV, 8, 128), jnp.float32),
            pltpu.SemaphoreType.DMA((4,)),
            pltpu.SemaphoreType.DMA((4,)),
            pltpu.SemaphoreType.DMA((4,)),
            pltpu.SemaphoreType.DMA((4,)),
            pltpu.SemaphoreType.DMA((4,)),
            pltpu.SemaphoreType.DMA((4,)),
            pltpu.SemaphoreType.DMA(()),
        ],
        compiler_params=pltpu.CompilerParams(
            collective_id=0, vmem_limit_bytes=100 * 1024 * 1024),
    )(x, w_mat)
    return out
